# baseline (speedup 1.0000x reference)
import functools

import jax
import jax.numpy as jnp
from jax import lax
from jax.experimental import pallas as pl
from jax.experimental.pallas import tpu as pltpu
from jax.experimental.pallas import tpu_sc as plsc

N = 10000
E = 320000
SD = 128
MO = 8
H = 32
ROW = 8

NUM_CORES = 2
NUM_SUBCORES = 16
NW = NUM_CORES * NUM_SUBCORES
CHUNK = 1280
K = 4
CH = 8
GROUPS = CH // K
EPW = CH * CHUNK
NPAD = NW * ((N + NW - 1) // NW)
NPAD = 10240
ROWS_PER_SUB = NPAD // NUM_SUBCORES

_BLK = 2000


def _msg_body(x_ref, w1_ref, b1_ref, w2_ref, b2_ref, w3p_ref, b3r_ref,
              root_ref, rootb_ref, m_ref, r_ref):
    x = x_ref[...]
    h = jnp.maximum(jnp.dot(x, w1_ref[...], preferred_element_type=jnp.float32, precision=lax.Precision.HIGHEST)
                    + b1_ref[...], 0.0)
    h = jnp.maximum(jnp.dot(h, w2_ref[...], preferred_element_type=jnp.float32, precision=lax.Precision.HIGHEST)
                    + b2_ref[...], 0.0)
    t = jnp.dot(x, w3p_ref[...], preferred_element_type=jnp.float32, precision=lax.Precision.HIGHEST)
    jj = lax.broadcasted_iota(jnp.int32, (H, H * MO), 1)
    kk = lax.broadcasted_iota(jnp.int32, (H, H * MO), 0)
    sel = (jj // MO == kk).astype(jnp.float32)
    jo = lax.broadcasted_iota(jnp.int32, (H * MO, MO), 0)
    oo = lax.broadcasted_iota(jnp.int32, (H * MO, MO), 1)
    col = (jo % MO == oo).astype(jnp.float32)
    hrep = jnp.dot(h, sel, preferred_element_type=jnp.float32, precision=lax.Precision.HIGHEST)
    msg = (jnp.dot(hrep * t, col, preferred_element_type=jnp.float32, precision=lax.Precision.HIGHEST)
           + jnp.dot(x, b3r_ref[...], preferred_element_type=jnp.float32, precision=lax.Precision.HIGHEST))
    if ROW == MO:
        m_ref[...] = msg
    else:
        m_ref[...] = jnp.concatenate(
            [msg, jnp.zeros((msg.shape[0], ROW - MO), jnp.float32)], axis=1)
    r_ref[...] = (jnp.dot(x, root_ref[...], preferred_element_type=jnp.float32, precision=lax.Precision.HIGHEST)
                  + rootb_ref[...])


def _node_messages(x, w1, b1, w2, b2, w3p, b3r, root, rootb):
    grid = (N // _BLK,)
    full = lambda shape: pl.BlockSpec(shape, lambda i: (0, 0))
    return pl.pallas_call(
        _msg_body,
        grid=grid,
        in_specs=[
            pl.BlockSpec((_BLK, SD), lambda i: (i, 0)),
            full((SD, H)), full((1, H)),
            full((H, H)), full((1, H)),
            full((SD, H * MO)), full((SD, MO)),
            full((SD, MO)), full((1, MO)),
        ],
        out_specs=[
            pl.BlockSpec((_BLK, ROW), lambda i: (i, 0)),
            pl.BlockSpec((_BLK, MO), lambda i: (i, 0)),
        ],
        out_shape=[
            jax.ShapeDtypeStruct((N, ROW), jnp.float32),
            jax.ShapeDtypeStruct((N, MO), jnp.float32),
        ],
    )(x, w1, b1, w2, b2, w3p, b3r, root, rootb)


def _eidx_body(s_ref, o_ref):
    s = s_ref[...]
    o_ref[...] = s * ROW + lax.broadcasted_iota(jnp.int32, (s.shape[0], ROW), 1)


def _edge_elem_indices(src_flat):
    b = 2048
    n = src_flat.shape[0] // b
    return pl.pallas_call(
        _eidx_body,
        grid=(n,),
        in_specs=[pl.BlockSpec((b, 1), lambda i: (i, 0))],
        out_specs=pl.BlockSpec((b, ROW), lambda i: (i, 0)),
        out_shape=jax.ShapeDtypeStruct((src_flat.shape[0], ROW), jnp.int32),
    )(src_flat.reshape(-1, 1))


def _edge_aggregate_half(m_flat, eidx_p, dst_p, zeros_init):
    mesh = plsc.VectorSubcoreMesh(core_axis_name="c", subcore_axis_name="s",
                                  num_cores=1)

    @functools.partial(
        pl.kernel,
        out_type=jax.ShapeDtypeStruct((NPAD, ROW), jnp.float32),
        mesh=mesh,
        scratch_types=[
            pltpu.VMEM((CHUNK * ROW,), jnp.int32),
            pltpu.VMEM((CH, CHUNK), jnp.int32),
            pltpu.VMEM((N * ROW,), jnp.float32),
            pltpu.VMEM((CHUNK, ROW), jnp.float32),
            pltpu.VMEM_SHARED((NPAD, ROW), jnp.float32),
            pltpu.SemaphoreType.DMA,
        ],
        compiler_params=pltpu.CompilerParams(use_tc_tiling_on_sc=False,
                                             needs_layout_passes=False),
    )
    def edge_kernel(m_hbm, eidx_hbm, dst_hbm, zero_hbm, out_hbm,
                    eidx_v, dst_v, m_tile, buf_v, agg_sh, sem):
        sid = lax.axis_index("s")
        row0 = sid * ROWS_PER_SUB
        pltpu.sync_copy(zero_hbm.at[pl.ds(row0, ROWS_PER_SUB)],
                        agg_sh.at[pl.ds(row0, ROWS_PER_SUB)])
        pltpu.sync_copy(m_hbm, m_tile)
        pltpu.sync_copy(dst_hbm.at[sid], dst_v)
        plsc.subcore_barrier()

        lanes = lax.broadcasted_iota(jnp.int32, (16,), 0)
        hi = lax.shift_right_logical(lanes, 3)
        lo = lax.bitwise_and(lanes, 7)

        @pl.loop(0, CH)
        def _(c):
            pltpu.sync_copy(eidx_hbm.at[sid, c], eidx_v)

            @plsc.parallel_loop(0, CHUNK // 2, unroll=8)
            def _(v):
                ev = eidx_v[pl.ds(v * 16, 16)]
                vals = plsc.load_gather(m_tile, [ev])
                plsc.store_scatter(buf_v, [2 * v + hi, lo], vals)

            pltpu.sync_copy(buf_v, agg_sh.at[dst_v.at[c]], add=True)

        plsc.subcore_barrier()
        pltpu.sync_copy(agg_sh.at[pl.ds(row0, ROWS_PER_SUB)],
                        out_hbm.at[pl.ds(row0, ROWS_PER_SUB)])

    return edge_kernel(m_flat, eidx_p, dst_p, zeros_init)


def _update_body(x_ref, p0_ref, p1_ref, r_ref, w1x_ref, w1m_ref, b1_ref,
                 w2_ref, b2_ref, w3_ref, b3_ref, o_ref):
    x = x_ref[...]
    msgs = p0_ref[:, :MO] + p1_ref[:, :MO] + r_ref[...]
    h = jnp.maximum(jnp.dot(x, w1x_ref[...], preferred_element_type=jnp.float32, precision=lax.Precision.HIGHEST)
                    + jnp.dot(msgs, w1m_ref[...], preferred_element_type=jnp.float32, precision=lax.Precision.HIGHEST)
                    + b1_ref[...], 0.0)
    h = jnp.maximum(jnp.dot(h, w2_ref[...], preferred_element_type=jnp.float32, precision=lax.Precision.HIGHEST)
                    + b2_ref[...], 0.0)
    o_ref[...] = (jnp.dot(h, w3_ref[...], preferred_element_type=jnp.float32, precision=lax.Precision.HIGHEST)
                  + b3_ref[...])


def _node_update(x, p0, p1, r, w1x, w1m, b1, w2, b2, w3, b3):
    grid = (N // _BLK,)
    full = lambda shape: pl.BlockSpec(shape, lambda i: (0, 0))
    return pl.pallas_call(
        _update_body,
        grid=grid,
        in_specs=[
            pl.BlockSpec((_BLK, SD), lambda i: (i, 0)),
            pl.BlockSpec((_BLK, ROW), lambda i: (i, 0)),
            pl.BlockSpec((_BLK, ROW), lambda i: (i, 0)),
            pl.BlockSpec((_BLK, MO), lambda i: (i, 0)),
            full((SD, H)), full((MO, H)), full((1, H)),
            full((H, H)), full((1, H)),
            full((H, SD)), full((1, SD)),
        ],
        out_specs=pl.BlockSpec((_BLK, SD), lambda i: (i, 0)),
        out_shape=jax.ShapeDtypeStruct((N, SD), jnp.float32),
    )(x, p0, p1, r, w1x, w1m, b1, w2, b2, w3, b3)


def kernel(x, edge_index, mW1, mb1, mW2, mb2, mW3, mb3, root, root_b,
           oW1, ob1, oW2, ob2, oW3, ob3):
    src = edge_index[0]
    dst = edge_index[1]
    pad = NW * EPW - E
    src_flat = jnp.concatenate([src, jnp.zeros((pad,), jnp.int32)])
    dst_p = jnp.concatenate([dst, jnp.full((pad,), N, jnp.int32)]).reshape(
        2, NW // 2, CH, CHUNK)
    eidx_p = _edge_elem_indices(src_flat).reshape(2, NW // 2, CH, CHUNK * ROW)

    mW3p = mW3.reshape(H, SD, MO).transpose(1, 0, 2).reshape(SD, H * MO)
    mb3r = mb3.reshape(SD, MO)
    zeros_init = jnp.zeros((NPAD, ROW), jnp.float32)

    m_tab, r = _node_messages(
        x, mW1, mb1.reshape(1, H), mW2, mb2.reshape(1, H),
        mW3p, mb3r, root, root_b.reshape(1, MO))
    m_flat = m_tab.reshape(N * ROW)
    p0 = _edge_aggregate_half(m_flat, eidx_p[0], dst_p[0], zeros_init)
    p1 = _edge_aggregate_half(m_flat, eidx_p[1], dst_p[1], zeros_init)
    return _node_update(
        x, p0, p1, r, oW1[:SD], oW1[SD:], ob1.reshape(1, H),
        oW2, ob2.reshape(1, H), oW3, ob3.reshape(1, SD))


# speedup vs baseline: 4.6473x; 4.6473x over previous
import functools

import jax
import jax.numpy as jnp
from jax import lax
from jax.experimental import pallas as pl
from jax.experimental.pallas import tpu as pltpu
from jax.experimental.pallas import tpu_sc as plsc

N = 10000
E = 320000
SD = 128
MO = 8
H = 32
ROW = 8

NUM_CORES = 2
NUM_SUBCORES = 16
NW = NUM_CORES * NUM_SUBCORES
CHUNK = 1280
K = 4
CH = 8
GROUPS = CH // K
EPW = CH * CHUNK
NPAD = 10240
ROWS_PER_SUB = NPAD // NUM_SUBCORES

_BLK = 2000


def _msg_body(x_ref, w1_ref, b1_ref, w2_ref, b2_ref, w3p_ref, b3r_ref,
              root_ref, rootb_ref, m_ref, r_ref):
    x = x_ref[...]
    h = jnp.maximum(jnp.dot(x, w1_ref[...], preferred_element_type=jnp.float32,
                            precision=lax.Precision.HIGHEST) + b1_ref[...], 0.0)
    h = jnp.maximum(jnp.dot(h, w2_ref[...], preferred_element_type=jnp.float32,
                            precision=lax.Precision.HIGHEST) + b2_ref[...], 0.0)
    t = jnp.dot(x, w3p_ref[...], preferred_element_type=jnp.float32,
                precision=lax.Precision.HIGHEST)
    jj = lax.broadcasted_iota(jnp.int32, (H, H * MO), 1)
    kk = lax.broadcasted_iota(jnp.int32, (H, H * MO), 0)
    sel = (jj // MO == kk).astype(jnp.float32)
    jo = lax.broadcasted_iota(jnp.int32, (H * MO, MO), 0)
    oo = lax.broadcasted_iota(jnp.int32, (H * MO, MO), 1)
    col = (jo % MO == oo).astype(jnp.float32)
    hrep = jnp.dot(h, sel, preferred_element_type=jnp.float32,
                   precision=lax.Precision.HIGHEST)
    msg = (jnp.dot(hrep * t, col, preferred_element_type=jnp.float32,
                   precision=lax.Precision.HIGHEST)
           + jnp.dot(x, b3r_ref[...], preferred_element_type=jnp.float32,
                     precision=lax.Precision.HIGHEST))
    m_ref[...] = msg
    r_ref[...] = (jnp.dot(x, root_ref[...], preferred_element_type=jnp.float32,
                          precision=lax.Precision.HIGHEST) + rootb_ref[...])


def _node_messages(x, w1, b1, w2, b2, w3p, b3r, root, rootb):
    grid = (N // _BLK,)
    full = lambda shape: pl.BlockSpec(shape, lambda i: (0, 0))
    return pl.pallas_call(
        _msg_body,
        grid=grid,
        in_specs=[
            pl.BlockSpec((_BLK, SD), lambda i: (i, 0)),
            full((SD, H)), full((1, H)),
            full((H, H)), full((1, H)),
            full((SD, H * MO)), full((SD, MO)),
            full((SD, MO)), full((1, MO)),
        ],
        out_specs=[
            pl.BlockSpec((_BLK, ROW), lambda i: (i, 0)),
            pl.BlockSpec((_BLK, MO), lambda i: (i, 0)),
        ],
        out_shape=[
            jax.ShapeDtypeStruct((N, ROW), jnp.float32),
            jax.ShapeDtypeStruct((N, MO), jnp.float32),
        ],
    )(x, w1, b1, w2, b2, w3p, b3r, root, rootb)


def _edge_aggregate(m_tab, src_p, dst_p, zeros_init):
    mesh = plsc.VectorSubcoreMesh(core_axis_name="c", subcore_axis_name="s")

    @functools.partial(
        pl.kernel,
        out_type=jax.ShapeDtypeStruct((NUM_CORES, NPAD, ROW), jnp.float32),
        mesh=mesh,
        scratch_types=[
            pltpu.VMEM((CH, CHUNK), jnp.int32),
            pltpu.VMEM((CH, CHUNK), jnp.int32),
            pltpu.VMEM((K * CHUNK, ROW), jnp.float32),
            pltpu.VMEM_SHARED((N, ROW), jnp.float32),
            pltpu.VMEM_SHARED((NPAD, ROW), jnp.float32),
            pltpu.SemaphoreType.DMA,
            pltpu.SemaphoreType.DMA,
        ],
        compiler_params=pltpu.CompilerParams(use_tc_tiling_on_sc=False),
    )
    def edge_kernel(m_hbm, src_hbm, dst_hbm, zero_hbm, out_hbm,
                    src_v, dst_v, msg_v, m_sh, agg_sh, gsem, ssem):
        cid = lax.axis_index("c")
        sid = lax.axis_index("s")
        wid = sid * NUM_CORES + cid
        row0 = sid * ROWS_PER_SUB
        pltpu.sync_copy(zero_hbm.at[pl.ds(row0, ROWS_PER_SUB)],
                        agg_sh.at[pl.ds(row0, ROWS_PER_SUB)])
        mrows = N // NUM_SUBCORES
        pltpu.sync_copy(m_hbm.at[pl.ds(sid * mrows, mrows)],
                        m_sh.at[pl.ds(sid * mrows, mrows)])
        pltpu.sync_copy(src_hbm.at[wid], src_v)
        pltpu.sync_copy(dst_hbm.at[wid], dst_v)
        plsc.subcore_barrier()

        @pl.loop(0, GROUPS)
        def _(g):
            base = g * K
            gathers = [
                pltpu.async_copy(
                    m_sh.at[src_v.at[base + b]],
                    msg_v.at[pl.ds(b * CHUNK, CHUNK)], gsem)
                for b in range(K)
            ]
            for h in gathers:
                h.wait()
            scatters = [
                pltpu.async_copy(
                    msg_v.at[pl.ds(b * CHUNK, CHUNK)],
                    agg_sh.at[dst_v.at[base + b]], ssem, add=True)
                for b in range(K)
            ]
            for h in scatters:
                h.wait()

        plsc.subcore_barrier()
        pltpu.sync_copy(agg_sh.at[pl.ds(row0, ROWS_PER_SUB)],
                        out_hbm.at[cid, pl.ds(row0, ROWS_PER_SUB)])

    return edge_kernel(m_tab, src_p, dst_p, zeros_init)


def _update_body(x_ref, p_ref, r_ref, w1x_ref, w1m_ref, b1_ref,
                 w2_ref, b2_ref, w3_ref, b3_ref, o_ref):
    x = x_ref[...]
    msgs = p_ref[0, :, :MO] + p_ref[1, :, :MO] + r_ref[...]
    h = jnp.maximum(jnp.dot(x, w1x_ref[...], preferred_element_type=jnp.float32,
                            precision=lax.Precision.HIGHEST)
                    + jnp.dot(msgs, w1m_ref[...],
                              preferred_element_type=jnp.float32,
                              precision=lax.Precision.HIGHEST)
                    + b1_ref[...], 0.0)
    h = jnp.maximum(jnp.dot(h, w2_ref[...], preferred_element_type=jnp.float32,
                            precision=lax.Precision.HIGHEST) + b2_ref[...], 0.0)
    o_ref[...] = (jnp.dot(h, w3_ref[...], preferred_element_type=jnp.float32,
                          precision=lax.Precision.HIGHEST) + b3_ref[...])


def _node_update(x, parts, r, w1x, w1m, b1, w2, b2, w3, b3):
    grid = (N // _BLK,)
    full = lambda shape: pl.BlockSpec(shape, lambda i: (0, 0))
    return pl.pallas_call(
        _update_body,
        grid=grid,
        in_specs=[
            pl.BlockSpec((_BLK, SD), lambda i: (i, 0)),
            pl.BlockSpec((NUM_CORES, _BLK, ROW), lambda i: (0, i, 0)),
            pl.BlockSpec((_BLK, MO), lambda i: (i, 0)),
            full((SD, H)), full((MO, H)), full((1, H)),
            full((H, H)), full((1, H)),
            full((H, SD)), full((1, SD)),
        ],
        out_specs=pl.BlockSpec((_BLK, SD), lambda i: (i, 0)),
        out_shape=jax.ShapeDtypeStruct((N, SD), jnp.float32),
    )(x, parts, r, w1x, w1m, b1, w2, b2, w3, b3)


def kernel(x, edge_index, mW1, mb1, mW2, mb2, mW3, mb3, root, root_b,
           oW1, ob1, oW2, ob2, oW3, ob3):
    src = edge_index[0]
    dst = edge_index[1]
    pad = NW * EPW - E
    src_p = jnp.concatenate([src, jnp.zeros((pad,), jnp.int32)]).reshape(
        NW, CH, CHUNK)
    dst_p = jnp.concatenate([dst, jnp.full((pad,), N, jnp.int32)]).reshape(
        NW, CH, CHUNK)

    mW3p = mW3.reshape(H, SD, MO).transpose(1, 0, 2).reshape(SD, H * MO)
    mb3r = mb3.reshape(SD, MO)
    zeros_init = jnp.zeros((NPAD, ROW), jnp.float32)

    m_tab, r = _node_messages(
        x, mW1, mb1.reshape(1, H), mW2, mb2.reshape(1, H),
        mW3p, mb3r, root, root_b.reshape(1, MO))
    parts = _edge_aggregate(m_tab, src_p, dst_p, zeros_init)
    return _node_update(
        x, parts, r, oW1[:SD], oW1[SD:], ob1.reshape(1, H),
        oW2, ob2.reshape(1, H), oW3, ob3.reshape(1, SD))


# speedup vs baseline: 4.7600x; 1.0243x over previous
import functools

import jax
import jax.numpy as jnp
from jax import lax
from jax.experimental import pallas as pl
from jax.experimental.pallas import tpu as pltpu
from jax.experimental.pallas import tpu_sc as plsc

N = 10000
E = 320000
SD = 128
MO = 8
H = 32
ROW = 8

NUM_CORES = 2
NUM_SUBCORES = 16
NW = NUM_CORES * NUM_SUBCORES
CHUNK = 1280
K = 4
CH = 8
GROUPS = CH // K
EPW = CH * CHUNK
NPAD = 10240
ROWS_PER_SUB = NPAD // NUM_SUBCORES

_BLK = 2000


def _msg_body(x_ref, w1_ref, b1_ref, w2_ref, b2_ref, w3p_ref, b3r_ref,
              root_ref, rootb_ref, m_ref, r_ref):
    x = x_ref[...]
    h = jnp.maximum(jnp.dot(x, w1_ref[...], preferred_element_type=jnp.float32,
                            precision=lax.Precision.HIGHEST) + b1_ref[...], 0.0)
    h = jnp.maximum(jnp.dot(h, w2_ref[...], preferred_element_type=jnp.float32,
                            precision=lax.Precision.HIGHEST) + b2_ref[...], 0.0)
    t = jnp.dot(x, w3p_ref[...], preferred_element_type=jnp.float32,
                precision=lax.Precision.HIGHEST)
    jj = lax.broadcasted_iota(jnp.int32, (H, H * MO), 1)
    kk = lax.broadcasted_iota(jnp.int32, (H, H * MO), 0)
    sel = (jj // MO == kk).astype(jnp.float32)
    jo = lax.broadcasted_iota(jnp.int32, (H * MO, MO), 0)
    oo = lax.broadcasted_iota(jnp.int32, (H * MO, MO), 1)
    col = (jo % MO == oo).astype(jnp.float32)
    hrep = jnp.dot(h, sel, preferred_element_type=jnp.float32,
                   precision=lax.Precision.HIGHEST)
    msg = (jnp.dot(hrep * t, col, preferred_element_type=jnp.float32,
                   precision=lax.Precision.HIGHEST)
           + jnp.dot(x, b3r_ref[...], preferred_element_type=jnp.float32,
                     precision=lax.Precision.HIGHEST))
    m_ref[...] = msg
    r_ref[...] = (jnp.dot(x, root_ref[...], preferred_element_type=jnp.float32,
                          precision=lax.Precision.HIGHEST) + rootb_ref[...])


def _node_messages(x, w1, b1, w2, b2, w3p, b3r, root, rootb):
    grid = (N // _BLK,)
    full = lambda shape: pl.BlockSpec(shape, lambda i: (0, 0))
    return pl.pallas_call(
        _msg_body,
        grid=grid,
        in_specs=[
            pl.BlockSpec((_BLK, SD), lambda i: (i, 0)),
            full((SD, H)), full((1, H)),
            full((H, H)), full((1, H)),
            full((SD, H * MO)), full((SD, MO)),
            full((SD, MO)), full((1, MO)),
        ],
        out_specs=[
            pl.BlockSpec((_BLK, ROW), lambda i: (i, 0)),
            pl.BlockSpec((_BLK, MO), lambda i: (i, 0)),
        ],
        out_shape=[
            jax.ShapeDtypeStruct((N, ROW), jnp.float32),
            jax.ShapeDtypeStruct((N, MO), jnp.float32),
        ],
    )(x, w1, b1, w2, b2, w3p, b3r, root, rootb)


def _edge_aggregate(m_tab, src_p, dst_p, zeros_init):
    mesh = plsc.VectorSubcoreMesh(core_axis_name="c", subcore_axis_name="s")

    @functools.partial(
        pl.kernel,
        out_type=jax.ShapeDtypeStruct((NUM_CORES, NPAD, ROW), jnp.float32),
        mesh=mesh,
        scratch_types=[
            pltpu.VMEM((CH, CHUNK), jnp.int32),
            pltpu.VMEM((CH, CHUNK), jnp.int32),
            pltpu.VMEM((GROUPS * K * CHUNK, ROW), jnp.float32),
            pltpu.VMEM_SHARED((N, ROW), jnp.float32),
            pltpu.VMEM_SHARED((NPAD, ROW), jnp.float32),
            pltpu.SemaphoreType.DMA,
            pltpu.SemaphoreType.DMA,
        ],
        compiler_params=pltpu.CompilerParams(use_tc_tiling_on_sc=False),
    )
    def edge_kernel(m_hbm, src_hbm, dst_hbm, zero_hbm, out_hbm,
                    src_v, dst_v, msg_v, m_sh, agg_sh, gsem, ssem):
        cid = lax.axis_index("c")
        sid = lax.axis_index("s")
        wid = sid * NUM_CORES + cid
        row0 = sid * ROWS_PER_SUB
        pltpu.sync_copy(zero_hbm.at[pl.ds(row0, ROWS_PER_SUB)],
                        agg_sh.at[pl.ds(row0, ROWS_PER_SUB)])
        mrows = N // NUM_SUBCORES
        pltpu.sync_copy(m_hbm.at[pl.ds(sid * mrows, mrows)],
                        m_sh.at[pl.ds(sid * mrows, mrows)])
        pltpu.sync_copy(src_hbm.at[wid], src_v)
        pltpu.sync_copy(dst_hbm.at[wid], dst_v)
        plsc.subcore_barrier()

        gathers = {0: [
            pltpu.async_copy(
                m_sh.at[src_v.at[b]],
                msg_v.at[pl.ds(b * CHUNK, CHUNK)], gsem)
            for b in range(K)
        ]}
        scatters = []
        for g in range(GROUPS):
            base = g * K
            for h in gathers.pop(g):
                h.wait()
            if g + 1 < GROUPS:
                nbase = (g + 1) * K
                gathers[g + 1] = [
                    pltpu.async_copy(
                        m_sh.at[src_v.at[nbase + b]],
                        msg_v.at[pl.ds((nbase + b) * CHUNK, CHUNK)], gsem)
                    for b in range(K)
                ]
            scatters += [
                pltpu.async_copy(
                    msg_v.at[pl.ds((base + b) * CHUNK, CHUNK)],
                    agg_sh.at[dst_v.at[base + b]], ssem, add=True)
                for b in range(K)
            ]
        for h in scatters:
            h.wait()

        plsc.subcore_barrier()
        pltpu.sync_copy(agg_sh.at[pl.ds(row0, ROWS_PER_SUB)],
                        out_hbm.at[cid, pl.ds(row0, ROWS_PER_SUB)])

    return edge_kernel(m_tab, src_p, dst_p, zeros_init)


def _update_body(x_ref, p_ref, r_ref, w1x_ref, w1m_ref, b1_ref,
                 w2_ref, b2_ref, w3_ref, b3_ref, o_ref):
    x = x_ref[...]
    msgs = p_ref[0, :, :MO] + p_ref[1, :, :MO] + r_ref[...]
    h = jnp.maximum(jnp.dot(x, w1x_ref[...], preferred_element_type=jnp.float32,
                            precision=lax.Precision.HIGHEST)
                    + jnp.dot(msgs, w1m_ref[...],
                              preferred_element_type=jnp.float32,
                              precision=lax.Precision.HIGHEST)
                    + b1_ref[...], 0.0)
    h = jnp.maximum(jnp.dot(h, w2_ref[...], preferred_element_type=jnp.float32,
                            precision=lax.Precision.HIGHEST) + b2_ref[...], 0.0)
    o_ref[...] = (jnp.dot(h, w3_ref[...], preferred_element_type=jnp.float32,
                          precision=lax.Precision.HIGHEST) + b3_ref[...])


def _node_update(x, parts, r, w1x, w1m, b1, w2, b2, w3, b3):
    grid = (N // _BLK,)
    full = lambda shape: pl.BlockSpec(shape, lambda i: (0, 0))
    return pl.pallas_call(
        _update_body,
        grid=grid,
        in_specs=[
            pl.BlockSpec((_BLK, SD), lambda i: (i, 0)),
            pl.BlockSpec((NUM_CORES, _BLK, ROW), lambda i: (0, i, 0)),
            pl.BlockSpec((_BLK, MO), lambda i: (i, 0)),
            full((SD, H)), full((MO, H)), full((1, H)),
            full((H, H)), full((1, H)),
            full((H, SD)), full((1, SD)),
        ],
        out_specs=pl.BlockSpec((_BLK, SD), lambda i: (i, 0)),
        out_shape=jax.ShapeDtypeStruct((N, SD), jnp.float32),
    )(x, parts, r, w1x, w1m, b1, w2, b2, w3, b3)


def kernel(x, edge_index, mW1, mb1, mW2, mb2, mW3, mb3, root, root_b,
           oW1, ob1, oW2, ob2, oW3, ob3):
    src = edge_index[0]
    dst = edge_index[1]
    pad = NW * EPW - E
    src_p = jnp.concatenate([src, jnp.zeros((pad,), jnp.int32)]).reshape(
        NW, CH, CHUNK)
    dst_p = jnp.concatenate([dst, jnp.full((pad,), N, jnp.int32)]).reshape(
        NW, CH, CHUNK)

    mW3p = mW3.reshape(H, SD, MO).transpose(1, 0, 2).reshape(SD, H * MO)
    mb3r = mb3.reshape(SD, MO)
    zeros_init = jnp.zeros((NPAD, ROW), jnp.float32)

    m_tab, r = _node_messages(
        x, mW1, mb1.reshape(1, H), mW2, mb2.reshape(1, H),
        mW3p, mb3r, root, root_b.reshape(1, MO))
    parts = _edge_aggregate(m_tab, src_p, dst_p, zeros_init)
    return _node_update(
        x, parts, r, oW1[:SD], oW1[SD:], ob1.reshape(1, H),
        oW2, ob2.reshape(1, H), oW3, ob3.reshape(1, SD))
